# baseline (speedup 1.0000x reference)
import functools

import jax
import jax.numpy as jnp
from jax import lax
from jax.experimental import pallas as pl
from jax.experimental.pallas import tpu as pltpu
from jax.experimental.pallas import tpu_sc as plsc

N = 10000
D = 128
E = 320000
ONE_PLUS_EPS = 1.0 + 0.1

NC = 2
NS = 16
L = 16
NW = NC * NS
NP = 10240
SLICE = NP // NS
BATCH = 128
NBP = 79
EP = NW * NBP * BATCH

_MESH = plsc.VectorSubcoreMesh(core_axis_name="c", subcore_axis_name="s")


def _fill(ref, value, n):
    vec = jnp.full((L,), value, dtype=ref.dtype)
    for i in range(n // L):
        ref[pl.ds(i * L, L)] = vec


RB = 320
TAIL_ROWS = N - (NW - 1) * RB


@functools.partial(
    pl.kernel,
    out_type=(jax.ShapeDtypeStruct((D,), jnp.float32),
              jax.ShapeDtypeStruct((3 * NC * NP,), jnp.float32),
              jax.ShapeDtypeStruct((NC * NS * D,), jnp.float32)),
    mesh=_MESH,
    compiler_params=pltpu.CompilerParams(needs_layout_passes=False),
    scratch_types=[
        pltpu.VMEM((NBP, BATCH), jnp.int32),
        pltpu.VMEM((NBP, BATCH), jnp.int32),
        pltpu.VMEM((NP,), jnp.float32),
        pltpu.VMEM((NBP, BATCH), jnp.float32),
        pltpu.VMEM((SLICE,), jnp.float32),
        pltpu.VMEM((SLICE,), jnp.float32),
        pltpu.VMEM((SLICE,), jnp.float32),
        pltpu.VMEM((SLICE,), jnp.float32),
        pltpu.VMEM((RB, D), jnp.float32),
        pltpu.VMEM((RB,), jnp.float32),
        pltpu.VMEM((D,), jnp.float32),
        pltpu.VMEM((NC * NS * D,), jnp.float32),
        pltpu.VMEM_SHARED((NP,), jnp.float32),
        pltpu.VMEM_SHARED((NP,), jnp.float32),
        pltpu.SemaphoreType.DMA,
        pltpu.SemaphoreType.DMA,
        pltpu.SemaphoreType.REGULAR,
    ],
)
def _u_kernel(srcp, dstp, x, vp, q_out, xchg, qx,
              src_v, dst_v, w_v, vals_v, rem_v, loc_v, comb_v, zb_v,
              xb_v, vpb_v, qb_v, qall_v,
              w_sp, acc_sp, dsem, xsem_dma, xsem):
    c = lax.axis_index("c")
    s = lax.axis_index("s")
    wid = s * NC + c
    sl = pl.ds(s * SLICE, SLICE)
    r0 = wid * RB

    last = r0 + RB > N

    @pl.when(jnp.logical_not(last))
    def _():
        pltpu.async_copy(x.at[pl.ds(r0, RB)], xb_v, xsem_dma)
        pltpu.async_copy(vp.at[pl.ds(r0, RB)], vpb_v, xsem_dma)

    @pl.when(last)
    def _():
        pltpu.async_copy(x.at[pl.ds(r0, TAIL_ROWS)],
                         xb_v.at[pl.ds(0, TAIL_ROWS)], xsem_dma)
        pltpu.async_copy(vp.at[pl.ds(r0, RB)], vpb_v, xsem_dma)

    def xbarrier():
        plsc.subcore_barrier()

        @pl.when(s == 0)
        def _():
            pltpu.semaphore_signal(xsem, 1, core_index=1 - c)
            pltpu.semaphore_wait(xsem, 1)

        plsc.subcore_barrier()

    def scatter_fire_drain():
        def body(j, carry):
            pltpu.async_copy(vals_v.at[j], acc_sp.at[src_v.at[j]], dsem,
                             add=True)
            return carry

        lax.fori_loop(0, NBP, body, 0)

        def drain(j, carry):
            pltpu.make_async_copy(vals_v.at[j], acc_sp.at[src_v.at[j]],
                                  dsem).wait()
            return carry

        lax.fori_loop(0, NBP, drain, 0)

    _fill(zb_v, 0.0, SLICE)
    pltpu.sync_copy(zb_v, acc_sp.at[sl])
    pltpu.sync_copy(srcp.at[wid], src_v)
    pltpu.sync_copy(dstp.at[wid], dst_v)
    _fill(vals_v.at[0], 1.0, BATCH)

    def ones_body(j, carry):
        pltpu.async_copy(vals_v.at[0], acc_sp.at[src_v.at[j]], dsem, add=True)
        return carry

    def ones_drain(j, carry):
        pltpu.make_async_copy(vals_v.at[0], acc_sp.at[src_v.at[j]],
                              dsem).wait()
        return carry

    plsc.subcore_barrier()
    lax.fori_loop(0, NBP, ones_body, 0)
    lax.fori_loop(0, NBP, ones_drain, 0)
    plsc.subcore_barrier()
    pltpu.sync_copy(acc_sp.at[sl],
                    xchg.at[pl.ds(c * NP + s * SLICE, SLICE)])
    xbarrier()

    for p in range(2):
        pltpu.sync_copy(
            xchg.at[pl.ds((p * NC + (1 - c)) * NP + s * SLICE, SLICE)], rem_v)
        pltpu.sync_copy(acc_sp.at[sl], loc_v)
        for i in range(SLICE // L):
            ii = pl.ds(i * L, L)
            if p == 0:
                wprev = ONE_PLUS_EPS
            else:
                wprev = ONE_PLUS_EPS * w_v[pl.ds(s * SLICE + i * L, L)]
            comb_v[ii] = wprev + loc_v[ii] + rem_v[ii]
        pltpu.sync_copy(comb_v, w_sp.at[sl])
        pltpu.sync_copy(zb_v, acc_sp.at[sl])
        plsc.subcore_barrier()
        pltpu.sync_copy(w_sp, w_v)

        def gbody(j, carry):
            for t in range(BATCH // L):
                idx = dst_v[j, pl.ds(t * L, L)]
                vals_v[j, pl.ds(t * L, L)] = plsc.load_gather(w_v, [idx])
            pltpu.async_copy(vals_v.at[j], acc_sp.at[src_v.at[j]], dsem,
                             add=True)
            return carry

        lax.fori_loop(0, NBP, gbody, 0)

        def gdrain(j, carry):
            pltpu.make_async_copy(vals_v.at[j], acc_sp.at[src_v.at[j]],
                                  dsem).wait()
            return carry

        lax.fori_loop(0, NBP, gdrain, 0)
        plsc.subcore_barrier()
        pltpu.sync_copy(
            acc_sp.at[sl],
            xchg.at[pl.ds(((p + 1) * NC + c) * NP + s * SLICE, SLICE)])
        xbarrier()

    pltpu.sync_copy(xchg.at[pl.ds((2 * NC + (1 - c)) * NP + r0, RB)],
                    rem_v.at[pl.ds(0, RB)])
    pltpu.sync_copy(acc_sp.at[pl.ds(r0, RB)], loc_v.at[pl.ds(0, RB)])

    @pl.when(jnp.logical_not(last))
    def _():
        pltpu.make_async_copy(x.at[pl.ds(r0, RB)], xb_v, xsem_dma).wait()
        pltpu.make_async_copy(vp.at[pl.ds(r0, RB)], vpb_v, xsem_dma).wait()

    @pl.when(last)
    def _():
        pltpu.make_async_copy(x.at[pl.ds(r0, TAIL_ROWS)],
                              xb_v.at[pl.ds(0, TAIL_ROWS)], xsem_dma).wait()
        pltpu.make_async_copy(vp.at[pl.ds(r0, RB)], vpb_v, xsem_dma).wait()

    for i in range(RB // L):
        ii = pl.ds(i * L, L)
        comb_v[ii] = (ONE_PLUS_EPS * w_v[pl.ds(r0 + i * L, L)]
                      + loc_v[ii] + rem_v[ii]) * vpb_v[ii]

    nblk = jnp.minimum(RB, N - r0) // L

    def mblk(b, qs):
        cvec = comb_v[pl.ds(b * L, L)]
        for k in range(L):
            cv = jnp.full((L,), cvec[k], jnp.float32)
            i = b * L + k
            qs = tuple(qs[d] + cv * xb_v[i, pl.ds(d * L, L)]
                       for d in range(D // L))
        return qs

    qs = lax.fori_loop(0, nblk, mblk,
                       tuple(jnp.zeros((L,), jnp.float32)
                             for _ in range(D // L)))
    for d in range(D // L):
        qb_v[pl.ds(d * L, L)] = qs[d]
    pltpu.sync_copy(qb_v, qx.at[pl.ds((c * NS + s) * D, D)])
    xbarrier()

    @pl.when(jnp.logical_and(c == 0, s == 0))
    def _():
        pltpu.sync_copy(qx, qall_v)
        for d in range(D // L):
            acc = jnp.zeros((L,), jnp.float32)
            for r in range(NC * NS):
                acc = acc + qall_v[pl.ds(r * D + d * L, L)]
            qb_v[pl.ds(d * L, L)] = acc
        pltpu.sync_copy(qb_v, q_out)


def kernel(x, v, edge_index):
    src = edge_index[0]
    dst = edge_index[1]
    pad = EP - E
    srcp = jnp.concatenate([src, jnp.full((pad,), N, jnp.int32)])
    dstp = jnp.concatenate([dst, jnp.zeros((pad,), jnp.int32)])
    srcp = srcp.reshape(NW, NBP, BATCH)
    dstp = dstp.reshape(NW, NBP, BATCH)

    vp = jnp.pad(v, (0, NP - N))
    q, _, _ = _u_kernel(srcp, dstp, x, vp)
    return q


# speedup vs baseline: 1.0868x; 1.0868x over previous
import functools

import jax
import jax.numpy as jnp
from jax import lax
from jax.experimental import pallas as pl
from jax.experimental.pallas import tpu as pltpu
from jax.experimental.pallas import tpu_sc as plsc

N = 10000
D = 128
E = 320000
ONE_PLUS_EPS = 1.0 + 0.1

NC = 2
NS = 16
L = 16
NW = NC * NS
NP = 10240
SLICE = NP // NS
BATCH = 80
NBP = 125

_MESH = plsc.VectorSubcoreMesh(core_axis_name="c", subcore_axis_name="s")


def _fill(ref, value, n):
    vec = jnp.full((L,), value, dtype=ref.dtype)
    for i in range(n // L):
        ref[pl.ds(i * L, L)] = vec


RB = 320
TAIL_ROWS = N - (NW - 1) * RB


@functools.partial(
    pl.kernel,
    out_type=(jax.ShapeDtypeStruct((D,), jnp.float32),
              jax.ShapeDtypeStruct((3 * NC * NP,), jnp.float32),
              jax.ShapeDtypeStruct((NC * NS * D,), jnp.float32)),
    mesh=_MESH,
    compiler_params=pltpu.CompilerParams(needs_layout_passes=False),
    scratch_types=[
        pltpu.VMEM((NBP, BATCH), jnp.int32),
        pltpu.VMEM((NBP, BATCH), jnp.int32),
        pltpu.VMEM((NP,), jnp.float32),
        pltpu.VMEM((NBP, BATCH), jnp.float32),
        pltpu.VMEM((SLICE,), jnp.float32),
        pltpu.VMEM((SLICE,), jnp.float32),
        pltpu.VMEM((SLICE,), jnp.float32),
        pltpu.VMEM((SLICE,), jnp.float32),
        pltpu.VMEM((RB, D), jnp.float32),
        pltpu.VMEM((RB,), jnp.float32),
        pltpu.VMEM((D,), jnp.float32),
        pltpu.VMEM((NC * NS * D,), jnp.float32),
        pltpu.VMEM_SHARED((NP,), jnp.float32),
        pltpu.VMEM_SHARED((NP,), jnp.float32),
        pltpu.SemaphoreType.DMA,
        pltpu.SemaphoreType.DMA,
        pltpu.SemaphoreType.REGULAR,
    ],
)
def _u_kernel(srcp, dstp, x, vp, q_out, xchg, qx,
              src_v, dst_v, w_v, vals_v, rem_v, loc_v, comb_v, zb_v,
              xb_v, vpb_v, qb_v, qall_v,
              w_sp, acc_sp, dsem, xsem_dma, xsem):
    c = lax.axis_index("c")
    s = lax.axis_index("s")
    wid = s * NC + c
    sl = pl.ds(s * SLICE, SLICE)
    r0 = wid * RB

    last = r0 + RB > N

    @pl.when(jnp.logical_not(last))
    def _():
        pltpu.async_copy(x.at[pl.ds(r0, RB)], xb_v, xsem_dma)
        pltpu.async_copy(vp.at[pl.ds(r0, RB)], vpb_v, xsem_dma)

    @pl.when(last)
    def _():
        pltpu.async_copy(x.at[pl.ds(r0, TAIL_ROWS)],
                         xb_v.at[pl.ds(0, TAIL_ROWS)], xsem_dma)
        pltpu.async_copy(vp.at[pl.ds(r0, TAIL_ROWS)],
                         vpb_v.at[pl.ds(0, TAIL_ROWS)], xsem_dma)

    def xbarrier():
        plsc.subcore_barrier()

        @pl.when(s == 0)
        def _():
            pltpu.semaphore_signal(xsem, 1, core_index=1 - c)
            pltpu.semaphore_wait(xsem, 1)

        plsc.subcore_barrier()

    def scatter_fire_drain():
        def body(j, carry):
            pltpu.async_copy(vals_v.at[j], acc_sp.at[src_v.at[j]], dsem,
                             add=True)
            return carry

        lax.fori_loop(0, NBP, body, 0)

        def drain(j, carry):
            pltpu.make_async_copy(vals_v.at[j], acc_sp.at[src_v.at[j]],
                                  dsem).wait()
            return carry

        lax.fori_loop(0, NBP, drain, 0)

    _fill(zb_v, 0.0, SLICE)
    pltpu.sync_copy(zb_v, acc_sp.at[sl])
    pltpu.sync_copy(srcp.at[wid], src_v)
    pltpu.sync_copy(dstp.at[wid], dst_v)
    _fill(vals_v.at[0], 1.0, BATCH)

    def ones_body(j, carry):
        pltpu.async_copy(vals_v.at[0], acc_sp.at[src_v.at[j]], dsem, add=True)
        return carry

    def ones_drain(j, carry):
        pltpu.make_async_copy(vals_v.at[0], acc_sp.at[src_v.at[j]],
                              dsem).wait()
        return carry

    plsc.subcore_barrier()
    lax.fori_loop(0, NBP, ones_body, 0)
    lax.fori_loop(0, NBP, ones_drain, 0)
    plsc.subcore_barrier()
    pltpu.sync_copy(acc_sp.at[sl],
                    xchg.at[pl.ds(c * NP + s * SLICE, SLICE)])
    xbarrier()

    for p in range(2):
        pltpu.sync_copy(
            xchg.at[pl.ds((p * NC + (1 - c)) * NP + s * SLICE, SLICE)], rem_v)
        pltpu.sync_copy(acc_sp.at[sl], loc_v)
        for i in range(SLICE // L):
            ii = pl.ds(i * L, L)
            if p == 0:
                wprev = ONE_PLUS_EPS
            else:
                wprev = ONE_PLUS_EPS * w_v[pl.ds(s * SLICE + i * L, L)]
            comb_v[ii] = wprev + loc_v[ii] + rem_v[ii]
        pltpu.sync_copy(comb_v, w_sp.at[sl])
        pltpu.sync_copy(zb_v, acc_sp.at[sl])
        plsc.subcore_barrier()
        pltpu.sync_copy(w_sp, w_v)

        def gbody(j, carry):
            for t in range(BATCH // L):
                idx = dst_v[j, pl.ds(t * L, L)]
                vals_v[j, pl.ds(t * L, L)] = plsc.load_gather(w_v, [idx])
            pltpu.async_copy(vals_v.at[j], acc_sp.at[src_v.at[j]], dsem,
                             add=True)
            return carry

        lax.fori_loop(0, NBP, gbody, 0)

        def gdrain(j, carry):
            pltpu.make_async_copy(vals_v.at[j], acc_sp.at[src_v.at[j]],
                                  dsem).wait()
            return carry

        lax.fori_loop(0, NBP, gdrain, 0)
        plsc.subcore_barrier()
        pltpu.sync_copy(
            acc_sp.at[sl],
            xchg.at[pl.ds(((p + 1) * NC + c) * NP + s * SLICE, SLICE)])
        xbarrier()

    pltpu.sync_copy(xchg.at[pl.ds((2 * NC + (1 - c)) * NP + r0, RB)],
                    rem_v.at[pl.ds(0, RB)])
    pltpu.sync_copy(acc_sp.at[pl.ds(r0, RB)], loc_v.at[pl.ds(0, RB)])

    @pl.when(jnp.logical_not(last))
    def _():
        pltpu.make_async_copy(x.at[pl.ds(r0, RB)], xb_v, xsem_dma).wait()
        pltpu.make_async_copy(vp.at[pl.ds(r0, RB)], vpb_v, xsem_dma).wait()

    @pl.when(last)
    def _():
        pltpu.make_async_copy(x.at[pl.ds(r0, TAIL_ROWS)],
                              xb_v.at[pl.ds(0, TAIL_ROWS)], xsem_dma).wait()
        pltpu.make_async_copy(vp.at[pl.ds(r0, TAIL_ROWS)],
                              vpb_v.at[pl.ds(0, TAIL_ROWS)], xsem_dma).wait()

    for i in range(RB // L):
        ii = pl.ds(i * L, L)
        comb_v[ii] = (ONE_PLUS_EPS * w_v[pl.ds(r0 + i * L, L)]
                      + loc_v[ii] + rem_v[ii]) * vpb_v[ii]

    nblk = jnp.minimum(RB, N - r0) // L

    def mblk(b, qs):
        cvec = comb_v[pl.ds(b * L, L)]
        for k in range(L):
            cv = jnp.full((L,), cvec[k], jnp.float32)
            i = b * L + k
            qs = tuple(qs[d] + cv * xb_v[i, pl.ds(d * L, L)]
                       for d in range(D // L))
        return qs

    qs = lax.fori_loop(0, nblk, mblk,
                       tuple(jnp.zeros((L,), jnp.float32)
                             for _ in range(D // L)))
    for d in range(D // L):
        qb_v[pl.ds(d * L, L)] = qs[d]
    pltpu.sync_copy(qb_v, qx.at[pl.ds((c * NS + s) * D, D)])
    xbarrier()

    @pl.when(jnp.logical_and(c == 0, s == 0))
    def _():
        pltpu.sync_copy(qx, qall_v)
        for d in range(D // L):
            acc = jnp.zeros((L,), jnp.float32)
            for r in range(NC * NS):
                acc = acc + qall_v[pl.ds(r * D + d * L, L)]
            qb_v[pl.ds(d * L, L)] = acc
        pltpu.sync_copy(qb_v, q_out)


def kernel(x, v, edge_index):
    srcp = edge_index[0].reshape(NW, NBP, BATCH)
    dstp = edge_index[1].reshape(NW, NBP, BATCH)
    q, _, _ = _u_kernel(srcp, dstp, x, v)
    return q


# speedup vs baseline: 1.0888x; 1.0019x over previous
import functools

import jax
import jax.numpy as jnp
from jax import lax
from jax.experimental import pallas as pl
from jax.experimental.pallas import tpu as pltpu
from jax.experimental.pallas import tpu_sc as plsc

N = 10000
D = 128
E = 320000
ONE_PLUS_EPS = 1.0 + 0.1

NC = 2
NS = 16
L = 16
NW = NC * NS
NP = 10240
SLICE = NP // NS
BATCH = 80
NBP = 125

_MESH = plsc.VectorSubcoreMesh(core_axis_name="c", subcore_axis_name="s")


def _fill(ref, value, n):
    vec = jnp.full((L,), value, dtype=ref.dtype)
    for i in range(n // L):
        ref[pl.ds(i * L, L)] = vec


RB = 320
TAIL_ROWS = N - (NW - 1) * RB


@functools.partial(
    pl.kernel,
    out_type=(jax.ShapeDtypeStruct((D,), jnp.float32),
              jax.ShapeDtypeStruct((3 * NC * NP,), jnp.float32),
              jax.ShapeDtypeStruct((NC * D,), jnp.float32)),
    mesh=_MESH,
    compiler_params=pltpu.CompilerParams(needs_layout_passes=False),
    scratch_types=[
        pltpu.VMEM((NBP, BATCH), jnp.int32),
        pltpu.VMEM((NBP, BATCH), jnp.int32),
        pltpu.VMEM((NP,), jnp.float32),
        pltpu.VMEM((NBP, BATCH), jnp.float32),
        pltpu.VMEM((SLICE,), jnp.float32),
        pltpu.VMEM((SLICE,), jnp.float32),
        pltpu.VMEM((SLICE,), jnp.float32),
        pltpu.VMEM((SLICE,), jnp.float32),
        pltpu.VMEM((RB, D), jnp.float32),
        pltpu.VMEM((RB,), jnp.float32),
        pltpu.VMEM((D,), jnp.float32),
        pltpu.VMEM((NC * D,), jnp.float32),
        pltpu.VMEM((D,), jnp.int32),
        pltpu.VMEM_SHARED((NP,), jnp.float32),
        pltpu.VMEM_SHARED((NP,), jnp.float32),
        pltpu.VMEM_SHARED((D,), jnp.float32),
        pltpu.SemaphoreType.DMA,
        pltpu.SemaphoreType.DMA,
        pltpu.SemaphoreType.REGULAR,
    ],
)
def _u_kernel(srcp, dstp, x, vp, q_out, xchg, qx,
              src_v, dst_v, w_v, vals_v, rem_v, loc_v, comb_v, zb_v,
              xb_v, vpb_v, qb_v, qall_v, qidx_v,
              w_sp, acc_sp, q_sp, dsem, xsem_dma, xsem):
    c = lax.axis_index("c")
    s = lax.axis_index("s")
    wid = s * NC + c
    sl = pl.ds(s * SLICE, SLICE)
    r0 = wid * RB

    last = r0 + RB > N

    @pl.when(jnp.logical_not(last))
    def _():
        pltpu.async_copy(x.at[pl.ds(r0, RB)], xb_v, xsem_dma)
        pltpu.async_copy(vp.at[pl.ds(r0, RB)], vpb_v, xsem_dma)

    @pl.when(last)
    def _():
        pltpu.async_copy(x.at[pl.ds(r0, TAIL_ROWS)],
                         xb_v.at[pl.ds(0, TAIL_ROWS)], xsem_dma)
        pltpu.async_copy(vp.at[pl.ds(r0, TAIL_ROWS)],
                         vpb_v.at[pl.ds(0, TAIL_ROWS)], xsem_dma)

    def xbarrier():
        plsc.subcore_barrier()

        @pl.when(s == 0)
        def _():
            pltpu.semaphore_signal(xsem, 1, core_index=1 - c)
            pltpu.semaphore_wait(xsem, 1)

        plsc.subcore_barrier()

    def scatter_fire_drain():
        def body(j, carry):
            pltpu.async_copy(vals_v.at[j], acc_sp.at[src_v.at[j]], dsem,
                             add=True)
            return carry

        lax.fori_loop(0, NBP, body, 0)

        def drain(j, carry):
            pltpu.make_async_copy(vals_v.at[j], acc_sp.at[src_v.at[j]],
                                  dsem).wait()
            return carry

        lax.fori_loop(0, NBP, drain, 0)

    _fill(zb_v, 0.0, SLICE)
    pltpu.sync_copy(zb_v, acc_sp.at[sl])
    for d in range(D // L):
        qidx_v[pl.ds(d * L, L)] = lax.iota(jnp.int32, L) + d * L

    @pl.when(s == 0)
    def _():
        pltpu.sync_copy(zb_v.at[pl.ds(0, D)], q_sp)
    pltpu.sync_copy(srcp.at[wid], src_v)
    pltpu.sync_copy(dstp.at[wid], dst_v)
    _fill(vals_v.at[0], 1.0, BATCH)

    def ones_body(j, carry):
        pltpu.async_copy(vals_v.at[0], acc_sp.at[src_v.at[j]], dsem, add=True)
        return carry

    def ones_drain(j, carry):
        pltpu.make_async_copy(vals_v.at[0], acc_sp.at[src_v.at[j]],
                              dsem).wait()
        return carry

    plsc.subcore_barrier()
    lax.fori_loop(0, NBP, ones_body, 0)
    lax.fori_loop(0, NBP, ones_drain, 0)
    plsc.subcore_barrier()
    pltpu.sync_copy(acc_sp.at[sl],
                    xchg.at[pl.ds(c * NP + s * SLICE, SLICE)])
    xbarrier()

    for p in range(2):
        pltpu.sync_copy(
            xchg.at[pl.ds((p * NC + (1 - c)) * NP + s * SLICE, SLICE)], rem_v)
        pltpu.sync_copy(acc_sp.at[sl], loc_v)
        for i in range(SLICE // L):
            ii = pl.ds(i * L, L)
            if p == 0:
                wprev = ONE_PLUS_EPS
            else:
                wprev = ONE_PLUS_EPS * w_v[pl.ds(s * SLICE + i * L, L)]
            comb_v[ii] = wprev + loc_v[ii] + rem_v[ii]
        pltpu.sync_copy(comb_v, w_sp.at[sl])
        pltpu.sync_copy(zb_v, acc_sp.at[sl])
        plsc.subcore_barrier()
        pltpu.sync_copy(w_sp, w_v)

        def gbody(j, carry):
            for t in range(BATCH // L):
                idx = dst_v[j, pl.ds(t * L, L)]
                vals_v[j, pl.ds(t * L, L)] = plsc.load_gather(w_v, [idx])
            pltpu.async_copy(vals_v.at[j], acc_sp.at[src_v.at[j]], dsem,
                             add=True)
            return carry

        lax.fori_loop(0, NBP, gbody, 0)

        def gdrain(j, carry):
            pltpu.make_async_copy(vals_v.at[j], acc_sp.at[src_v.at[j]],
                                  dsem).wait()
            return carry

        lax.fori_loop(0, NBP, gdrain, 0)
        plsc.subcore_barrier()
        pltpu.sync_copy(
            acc_sp.at[sl],
            xchg.at[pl.ds(((p + 1) * NC + c) * NP + s * SLICE, SLICE)])
        xbarrier()

    pltpu.sync_copy(xchg.at[pl.ds((2 * NC + (1 - c)) * NP + r0, RB)],
                    rem_v.at[pl.ds(0, RB)])
    pltpu.sync_copy(acc_sp.at[pl.ds(r0, RB)], loc_v.at[pl.ds(0, RB)])

    @pl.when(jnp.logical_not(last))
    def _():
        pltpu.make_async_copy(x.at[pl.ds(r0, RB)], xb_v, xsem_dma).wait()
        pltpu.make_async_copy(vp.at[pl.ds(r0, RB)], vpb_v, xsem_dma).wait()

    @pl.when(last)
    def _():
        pltpu.make_async_copy(x.at[pl.ds(r0, TAIL_ROWS)],
                              xb_v.at[pl.ds(0, TAIL_ROWS)], xsem_dma).wait()
        pltpu.make_async_copy(vp.at[pl.ds(r0, TAIL_ROWS)],
                              vpb_v.at[pl.ds(0, TAIL_ROWS)], xsem_dma).wait()

    for i in range(RB // L):
        ii = pl.ds(i * L, L)
        comb_v[ii] = (ONE_PLUS_EPS * w_v[pl.ds(r0 + i * L, L)]
                      + loc_v[ii] + rem_v[ii]) * vpb_v[ii]

    nblk = jnp.minimum(RB, N - r0) // L
    ND = D // L

    def mblk(b, qsall):
        cvec = comb_v[pl.ds(b * L, L)]
        qsA = qsall[:ND]
        qsB = qsall[ND:]
        for k in range(0, L, 2):
            cvA = jnp.full((L,), cvec[k], jnp.float32)
            cvB = jnp.full((L,), cvec[k + 1], jnp.float32)
            iA = b * L + k
            iB = b * L + k + 1
            qsA = tuple(qsA[d] + cvA * xb_v[iA, pl.ds(d * L, L)]
                        for d in range(ND))
            qsB = tuple(qsB[d] + cvB * xb_v[iB, pl.ds(d * L, L)]
                        for d in range(ND))
        return qsA + qsB

    qs = lax.fori_loop(0, nblk, mblk,
                       tuple(jnp.zeros((L,), jnp.float32)
                             for _ in range(2 * ND)))
    for d in range(ND):
        qb_v[pl.ds(d * L, L)] = qs[d] + qs[d + ND]
    pltpu.sync_copy(qb_v, q_sp.at[qidx_v], add=True)
    plsc.subcore_barrier()

    @pl.when(s == 0)
    def _():
        pltpu.sync_copy(q_sp, qx.at[pl.ds(c * D, D)])

    xbarrier()

    @pl.when(jnp.logical_and(c == 0, s == 0))
    def _():
        pltpu.sync_copy(qx, qall_v)
        for d in range(ND):
            qb_v[pl.ds(d * L, L)] = (qall_v[pl.ds(d * L, L)]
                                     + qall_v[pl.ds(D + d * L, L)])
        pltpu.sync_copy(qb_v, q_out)


def kernel(x, v, edge_index):
    srcp = edge_index[0].reshape(NW, NBP, BATCH)
    dstp = edge_index[1].reshape(NW, NBP, BATCH)
    q, _, _ = _u_kernel(srcp, dstp, x, v)
    return q


# speedup vs baseline: 1.0916x; 1.0025x over previous
import functools

import jax
import jax.numpy as jnp
from jax import lax
from jax.experimental import pallas as pl
from jax.experimental.pallas import tpu as pltpu
from jax.experimental.pallas import tpu_sc as plsc

N = 10000
D = 128
E = 320000
ONE_PLUS_EPS = 1.0 + 0.1

NC = 2
NS = 16
L = 16
NW = NC * NS
NP = 10240
SLICE = NP // NS
BATCH = 80
NBP = 125

_MESH = plsc.VectorSubcoreMesh(core_axis_name="c", subcore_axis_name="s")


def _fill(ref, value, n):
    vec = jnp.full((L,), value, dtype=ref.dtype)
    for i in range(n // L):
        ref[pl.ds(i * L, L)] = vec


RB = 320
TAIL_ROWS = N - (NW - 1) * RB


@functools.partial(
    pl.kernel,
    out_type=(jax.ShapeDtypeStruct((D,), jnp.float32),
              jax.ShapeDtypeStruct((3 * NC * NP,), jnp.float32),
              jax.ShapeDtypeStruct((NC * D,), jnp.float32)),
    mesh=_MESH,
    compiler_params=pltpu.CompilerParams(needs_layout_passes=False),
    scratch_types=[
        pltpu.VMEM((NBP, BATCH), jnp.int32),
        pltpu.VMEM((NBP, BATCH), jnp.int32),
        pltpu.VMEM((NP,), jnp.float32),
        pltpu.VMEM((NBP, BATCH), jnp.float32),
        pltpu.VMEM((SLICE,), jnp.float32),
        pltpu.VMEM((SLICE,), jnp.float32),
        pltpu.VMEM((SLICE,), jnp.float32),
        pltpu.VMEM((SLICE,), jnp.float32),
        pltpu.VMEM((RB, D), jnp.float32),
        pltpu.VMEM((RB,), jnp.float32),
        pltpu.VMEM((D,), jnp.float32),
        pltpu.VMEM((NC * D,), jnp.float32),
        pltpu.VMEM((D,), jnp.int32),
        pltpu.VMEM_SHARED((NP,), jnp.float32),
        pltpu.VMEM_SHARED((NP,), jnp.float32),
        pltpu.VMEM_SHARED((D,), jnp.float32),
        pltpu.SemaphoreType.DMA,
        pltpu.SemaphoreType.DMA,
        pltpu.SemaphoreType.REGULAR,
    ],
)
def _u_kernel(srcp, dstp, x, vp, q_out, xchg, qx,
              src_v, dst_v, w_v, vals_v, rem_v, loc_v, comb_v, zb_v,
              xb_v, vpb_v, qb_v, qall_v, qidx_v,
              w_sp, acc_sp, q_sp, dsem, xsem_dma, xsem):
    c = lax.axis_index("c")
    s = lax.axis_index("s")
    wid = s * NC + c
    sl = pl.ds(s * SLICE, SLICE)
    r0 = wid * RB

    last = r0 + RB > N

    @pl.when(jnp.logical_not(last))
    def _():
        pltpu.async_copy(x.at[pl.ds(r0, RB)], xb_v, xsem_dma)
        pltpu.async_copy(vp.at[pl.ds(r0, RB)], vpb_v, xsem_dma)

    @pl.when(last)
    def _():
        pltpu.async_copy(x.at[pl.ds(r0, TAIL_ROWS)],
                         xb_v.at[pl.ds(0, TAIL_ROWS)], xsem_dma)
        pltpu.async_copy(vp.at[pl.ds(r0, TAIL_ROWS)],
                         vpb_v.at[pl.ds(0, TAIL_ROWS)], xsem_dma)

    def xbarrier():
        plsc.subcore_barrier()

        @pl.when(s == 0)
        def _():
            pltpu.semaphore_signal(xsem, 1, core_index=1 - c)
            pltpu.semaphore_wait(xsem, 1)

        plsc.subcore_barrier()

    _fill(zb_v, 0.0, SLICE)
    pltpu.sync_copy(zb_v, acc_sp.at[sl])
    for d in range(D // L):
        qidx_v[pl.ds(d * L, L)] = lax.iota(jnp.int32, L) + d * L

    @pl.when(s == 0)
    def _():
        pltpu.sync_copy(zb_v.at[pl.ds(0, D)], q_sp)
    pltpu.sync_copy(srcp.at[wid], src_v)
    pltpu.sync_copy(dstp.at[wid], dst_v)
    _fill(vals_v.at[0], 1.0, BATCH)

    def ones_body(j, carry):
        pltpu.async_copy(vals_v.at[0], acc_sp.at[src_v.at[j]], dsem, add=True)
        return carry

    def ones_drain(j, carry):
        pltpu.make_async_copy(vals_v.at[0], acc_sp.at[src_v.at[j]],
                              dsem).wait()
        return carry

    plsc.subcore_barrier()
    lax.fori_loop(0, NBP, ones_body, 0)
    lax.fori_loop(0, NBP, ones_drain, 0)
    plsc.subcore_barrier()
    pltpu.sync_copy(acc_sp.at[sl],
                    xchg.at[pl.ds(c * NP + s * SLICE, SLICE)])
    xbarrier()

    for p in range(2):
        pltpu.sync_copy(
            xchg.at[pl.ds((p * NC + (1 - c)) * NP + s * SLICE, SLICE)], rem_v)
        pltpu.sync_copy(acc_sp.at[sl], loc_v)
        for i in range(SLICE // L):
            ii = pl.ds(i * L, L)
            if p == 0:
                wprev = ONE_PLUS_EPS
            else:
                wprev = ONE_PLUS_EPS * w_v[pl.ds(s * SLICE + i * L, L)]
            comb_v[ii] = wprev + loc_v[ii] + rem_v[ii]
        pltpu.sync_copy(comb_v, w_sp.at[sl])
        pltpu.sync_copy(zb_v, acc_sp.at[sl])
        plsc.subcore_barrier()
        pltpu.sync_copy(w_sp, w_v)

        def gbody(j, carry):
            for t in range(BATCH // L):
                idx = dst_v[j, pl.ds(t * L, L)]
                vals_v[j, pl.ds(t * L, L)] = plsc.load_gather(w_v, [idx])
            pltpu.async_copy(vals_v.at[j], acc_sp.at[src_v.at[j]], dsem,
                             add=True)
            return carry

        lax.fori_loop(0, NBP, gbody, 0)

        def gdrain(j, carry):
            pltpu.make_async_copy(vals_v.at[j], acc_sp.at[src_v.at[j]],
                                  dsem).wait()
            return carry

        lax.fori_loop(0, NBP, gdrain, 0)
        plsc.subcore_barrier()
        pltpu.sync_copy(
            acc_sp.at[sl],
            xchg.at[pl.ds(((p + 1) * NC + c) * NP + s * SLICE, SLICE)])
        xbarrier()

    pltpu.sync_copy(xchg.at[pl.ds((2 * NC + (1 - c)) * NP + r0, RB)],
                    rem_v.at[pl.ds(0, RB)])
    pltpu.sync_copy(acc_sp.at[pl.ds(r0, RB)], loc_v.at[pl.ds(0, RB)])

    @pl.when(jnp.logical_not(last))
    def _():
        pltpu.make_async_copy(x.at[pl.ds(r0, RB)], xb_v, xsem_dma).wait()
        pltpu.make_async_copy(vp.at[pl.ds(r0, RB)], vpb_v, xsem_dma).wait()

    @pl.when(last)
    def _():
        pltpu.make_async_copy(x.at[pl.ds(r0, TAIL_ROWS)],
                              xb_v.at[pl.ds(0, TAIL_ROWS)], xsem_dma).wait()
        pltpu.make_async_copy(vp.at[pl.ds(r0, TAIL_ROWS)],
                              vpb_v.at[pl.ds(0, TAIL_ROWS)], xsem_dma).wait()

    for i in range(RB // L):
        ii = pl.ds(i * L, L)
        comb_v[ii] = (ONE_PLUS_EPS * w_v[pl.ds(r0 + i * L, L)]
                      + loc_v[ii] + rem_v[ii]) * vpb_v[ii]

    nblk = jnp.minimum(RB, N - r0) // L
    ND = D // L

    def mblk(b, qsall):
        cvec = comb_v[pl.ds(b * L, L)]
        qsA = qsall[:ND]
        qsB = qsall[ND:]
        for k in range(0, L, 2):
            cvA = jnp.full((L,), cvec[k], jnp.float32)
            cvB = jnp.full((L,), cvec[k + 1], jnp.float32)
            iA = b * L + k
            iB = b * L + k + 1
            qsA = tuple(qsA[d] + cvA * xb_v[iA, pl.ds(d * L, L)]
                        for d in range(ND))
            qsB = tuple(qsB[d] + cvB * xb_v[iB, pl.ds(d * L, L)]
                        for d in range(ND))
        return qsA + qsB

    qs = lax.fori_loop(0, nblk, mblk,
                       tuple(jnp.zeros((L,), jnp.float32)
                             for _ in range(2 * ND)))
    for d in range(ND):
        qb_v[pl.ds(d * L, L)] = qs[d] + qs[d + ND]
    pltpu.sync_copy(qb_v, q_sp.at[qidx_v], add=True)
    plsc.subcore_barrier()

    @pl.when(s == 0)
    def _():
        pltpu.sync_copy(q_sp, qx.at[pl.ds(c * D, D)])

    xbarrier()

    @pl.when(jnp.logical_and(c == 0, s == 0))
    def _():
        pltpu.sync_copy(qx, qall_v)
        for d in range(ND):
            qb_v[pl.ds(d * L, L)] = (qall_v[pl.ds(d * L, L)]
                                     + qall_v[pl.ds(D + d * L, L)])
        pltpu.sync_copy(qb_v, q_out)


def kernel(x, v, edge_index):
    srcp = edge_index[0].reshape(NW, NBP, BATCH)
    dstp = edge_index[1].reshape(NW, NBP, BATCH)
    q, _, _ = _u_kernel(srcp, dstp, x, v)
    return q
